# Initial kernel scaffold; baseline (speedup 1.0000x reference)
import jax
import jax.numpy as jnp
from jax.experimental import pallas as pl


def kernel(atomic_numbers, embedding_weight, electron_config, config_w, config_b):
    raise NotImplementedError("write your pallas kernel here")


# speedup vs baseline: 7.3401x; 7.3401x over previous
import functools

import jax
import jax.numpy as jnp
from jax import lax
from jax.experimental import pallas as pl
from jax.experimental.pallas import tpu as pltpu
from jax.experimental.pallas import tpu_sc as plsc

MAXZ1 = 102
NCFG = 20
D = 64
B = 4096
S = 200
N = B * S
TROWS = 208
NPAIR = N // 2
PROWS = TROWS * TROWS
NC = 2
NS = 16
NW = NC * NS
PER_W = NPAIR // NW
CHUNK = 256
NCHUNK = PER_W // CHUNK


def _build_table(embedding_weight, electron_config, config_w, config_b):
    emb_p = (jnp.zeros((TROWS, D), jnp.float32)
             .at[:MAXZ1].set(embedding_weight)
             .at[MAXZ1:2 * MAXZ1].set(embedding_weight))
    ec_p = jnp.zeros((TROWS, 128), jnp.float32).at[:MAXZ1, :NCFG].set(electron_config)
    wt_p = jnp.zeros((128, D), jnp.float32).at[:NCFG].set(config_w.T)
    b_p = jnp.broadcast_to(config_b[None, :], (8, D))
    bmask = (lax.broadcasted_iota(jnp.int32, (TROWS, 1), 0) < MAXZ1).astype(jnp.float32)
    def body(emb_ref, ec_ref, wt_ref, b_ref, m_ref, out_ref):
        elec = jnp.dot(ec_ref[...], wt_ref[...], preferred_element_type=jnp.float32)
        out_ref[...] = emb_ref[...] + elec + b_ref[0:1, :] * m_ref[...]
    return pl.pallas_call(
        body,
        out_shape=jax.ShapeDtypeStruct((TROWS, D), jnp.float32),
    )(emb_p, ec_p, wt_p, b_p, bmask)


def _pair_body(t_ref, out_ref):
    i = pl.program_id(0)
    row = t_ref[pl.ds(i, 1), :]
    out_ref[:, 0:D] = jnp.broadcast_to(row, (TROWS, D))
    out_ref[:, D:2 * D] = t_ref[...]


def _build_pair_table(table):
    return pl.pallas_call(
        _pair_body,
        grid=(TROWS,),
        in_specs=[pl.BlockSpec((TROWS, D), lambda i: (0, 0))],
        out_specs=pl.BlockSpec((TROWS, 2 * D), lambda i: (i, 0)),
        out_shape=jax.ShapeDtypeStruct((PROWS, 2 * D), jnp.float32),
    )(table)


def _pair_idx_body(even_ref, odd_ref, out_ref):
    col = lax.broadcasted_iota(jnp.int32, even_ref.shape, 1)
    adj_even = even_ref[...] + jnp.where(col == 0, MAXZ1, 0)
    out_ref[...] = adj_even * TROWS + odd_ref[...]


def _build_pair_indices(atomic_numbers):
    even = lax.slice(atomic_numbers, (0, 0), (B, S), (1, 2))
    odd = lax.slice(atomic_numbers, (0, 1), (B, S), (1, 2))
    rows = 512
    return pl.pallas_call(
        _pair_idx_body,
        grid=(B // rows,),
        in_specs=[pl.BlockSpec((rows, S // 2), lambda i: (i, 0)),
                  pl.BlockSpec((rows, S // 2), lambda i: (i, 0))],
        out_specs=pl.BlockSpec((rows, S // 2), lambda i: (i, 0)),
        out_shape=jax.ShapeDtypeStruct((B, S // 2), jnp.int32),
    )(even, odd)


def _gather_body(ptable_hbm, idx_hbm, out_hbm, idx_v, buf_v, sem):
    wid = lax.axis_index("s") * NC + lax.axis_index("c")
    base = wid * PER_W
    pltpu.sync_copy(idx_hbm.at[pl.ds(base, PER_W)], idx_v)
    def chunk(c, carry):
        pltpu.async_copy(ptable_hbm.at[idx_v.at[pl.ds(c * CHUNK, CHUNK)]],
                         buf_v, sem).wait()
        pltpu.sync_copy(buf_v, out_hbm.at[pl.ds(base + c * CHUNK, CHUNK)])
        return carry
    lax.fori_loop(0, NCHUNK, chunk, 0)


@functools.partial(
    pl.kernel,
    mesh=plsc.VectorSubcoreMesh(core_axis_name="c", subcore_axis_name="s"),
    out_type=jax.ShapeDtypeStruct((NPAIR, 2 * D), jnp.float32),
    scratch_types=[
        pltpu.VMEM((PER_W,), jnp.int32),
        pltpu.VMEM((CHUNK, 2 * D), jnp.float32),
        pltpu.SemaphoreType.DMA,
    ],
)
def _sc_gather(ptable_hbm, idx_hbm, out_hbm, idx_v, buf_v, sem):
    _gather_body(ptable_hbm, idx_hbm, out_hbm, idx_v, buf_v, sem)


def kernel(atomic_numbers, embedding_weight, electron_config, config_w, config_b):
    table = _build_table(embedding_weight, electron_config, config_w, config_b)
    ptable = _build_pair_table(table)
    pidx = _build_pair_indices(atomic_numbers).reshape(NPAIR)
    out = _sc_gather(ptable, pidx)
    return out.reshape(B, S, D)


# speedup vs baseline: 7.4700x; 1.0177x over previous
import functools

import jax
import jax.numpy as jnp
from jax import lax
from jax.experimental import pallas as pl
from jax.experimental.pallas import tpu as pltpu
from jax.experimental.pallas import tpu_sc as plsc

MAXZ1 = 102
NCFG = 20
D = 64
B = 4096
S = 200
N = B * S
TROWS = 208
NPAIR = N // 2
PROWS = TROWS * TROWS
NC = 2
NS = 16
NW = NC * NS
PER_W = NPAIR // NW
CHUNK = 400
NCHUNK = PER_W // CHUNK


def _build_table(embedding_weight, electron_config, config_w, config_b):
    emb_p = (jnp.zeros((TROWS, D), jnp.float32)
             .at[:MAXZ1].set(embedding_weight)
             .at[MAXZ1:2 * MAXZ1].set(embedding_weight))
    ec_p = jnp.zeros((TROWS, 128), jnp.float32).at[:MAXZ1, :NCFG].set(electron_config)
    wt_p = jnp.zeros((128, D), jnp.float32).at[:NCFG].set(config_w.T)
    b_p = jnp.broadcast_to(config_b[None, :], (8, D))
    bmask = (lax.broadcasted_iota(jnp.int32, (TROWS, 1), 0) < MAXZ1).astype(jnp.float32)
    def body(emb_ref, ec_ref, wt_ref, b_ref, m_ref, out_ref):
        elec = jnp.dot(ec_ref[...], wt_ref[...], preferred_element_type=jnp.float32)
        out_ref[...] = emb_ref[...] + elec + b_ref[0:1, :] * m_ref[...]
    return pl.pallas_call(
        body,
        out_shape=jax.ShapeDtypeStruct((TROWS, D), jnp.float32),
    )(emb_p, ec_p, wt_p, b_p, bmask)


def _pair_body(t_ref, out_ref):
    i = pl.program_id(0)
    row = t_ref[pl.ds(i, 1), :]
    out_ref[:, 0:D] = jnp.broadcast_to(row, (TROWS, D))
    out_ref[:, D:2 * D] = t_ref[...]


def _build_pair_table(table):
    return pl.pallas_call(
        _pair_body,
        grid=(TROWS,),
        in_specs=[pl.BlockSpec((TROWS, D), lambda i: (0, 0))],
        out_specs=pl.BlockSpec((TROWS, 2 * D), lambda i: (i, 0)),
        out_shape=jax.ShapeDtypeStruct((PROWS, 2 * D), jnp.float32),
    )(table)


def _pair_idx_body(even_ref, odd_ref, out_ref):
    col = lax.broadcasted_iota(jnp.int32, even_ref.shape, 1)
    adj_even = even_ref[...] + jnp.where(col == 0, MAXZ1, 0)
    out_ref[...] = adj_even * TROWS + odd_ref[...]


def _build_pair_indices(atomic_numbers):
    even = lax.slice(atomic_numbers, (0, 0), (B, S), (1, 2))
    odd = lax.slice(atomic_numbers, (0, 1), (B, S), (1, 2))
    rows = 512
    return pl.pallas_call(
        _pair_idx_body,
        grid=(B // rows,),
        in_specs=[pl.BlockSpec((rows, S // 2), lambda i: (i, 0)),
                  pl.BlockSpec((rows, S // 2), lambda i: (i, 0))],
        out_specs=pl.BlockSpec((rows, S // 2), lambda i: (i, 0)),
        out_shape=jax.ShapeDtypeStruct((B, S // 2), jnp.int32),
    )(even, odd)


def _gather_body(ptable_hbm, idx_hbm, out_hbm, idx_v,
                 buf0, buf1, gsem0, gsem1, wsem0, wsem1):
    wid = lax.axis_index("s") * NC + lax.axis_index("c")
    base = wid * PER_W
    pltpu.sync_copy(idx_hbm.at[pl.ds(base, PER_W)], idx_v)
    bufs = (buf0, buf1)
    gsems = (gsem0, gsem1)
    wsems = (wsem0, wsem1)

    def body(k, carry):
        handles = []
        for b in range(2):
            c = 2 * k + b
            @pl.when(k > 0)
            def _drain():
                pltpu.make_async_copy(
                    bufs[b], out_hbm.at[pl.ds(base, CHUNK)], wsems[b]).wait()
            handles.append(pltpu.async_copy(
                ptable_hbm.at[idx_v.at[pl.ds(c * CHUNK, CHUNK)]],
                bufs[b], gsems[b]))
        for b in range(2):
            c = 2 * k + b
            handles[b].wait()
            pltpu.async_copy(bufs[b],
                             out_hbm.at[pl.ds(base + c * CHUNK, CHUNK)],
                             wsems[b])
        return carry

    lax.fori_loop(0, NCHUNK // 2, body, 0)
    for b in range(2):
        pltpu.make_async_copy(
            bufs[b], out_hbm.at[pl.ds(base, CHUNK)], wsems[b]).wait()


@functools.partial(
    pl.kernel,
    mesh=plsc.VectorSubcoreMesh(core_axis_name="c", subcore_axis_name="s"),
    out_type=jax.ShapeDtypeStruct((NPAIR, 2 * D), jnp.float32),
    scratch_types=[
        pltpu.VMEM((PER_W,), jnp.int32),
        pltpu.VMEM((CHUNK, 2 * D), jnp.float32),
        pltpu.VMEM((CHUNK, 2 * D), jnp.float32),
        pltpu.SemaphoreType.DMA,
        pltpu.SemaphoreType.DMA,
        pltpu.SemaphoreType.DMA,
        pltpu.SemaphoreType.DMA,
    ],
)
def _sc_gather(ptable_hbm, idx_hbm, out_hbm, idx_v,
               buf0, buf1, gsem0, gsem1, wsem0, wsem1):
    _gather_body(ptable_hbm, idx_hbm, out_hbm, idx_v,
                 buf0, buf1, gsem0, gsem1, wsem0, wsem1)


def kernel(atomic_numbers, embedding_weight, electron_config, config_w, config_b):
    table = _build_table(embedding_weight, electron_config, config_w, config_b)
    ptable = _build_pair_table(table)
    pidx = _build_pair_indices(atomic_numbers).reshape(NPAIR)
    out = _sc_gather(ptable, pidx)
    return out.reshape(B, S, D)


# speedup vs baseline: 8.1595x; 1.0923x over previous
import functools

import jax
import jax.numpy as jnp
from jax import lax
from jax.experimental import pallas as pl
from jax.experimental.pallas import tpu as pltpu
from jax.experimental.pallas import tpu_sc as plsc

MAXZ1 = 102
NCFG = 20
D = 64
B = 4096
S = 200
N = B * S
TROWS = 208
NPAIR = N // 2
PROWS = TROWS * TROWS
NC = 2
NS = 16
NW = NC * NS
PER_W = NPAIR // NW
CHUNK = 400
NCHUNK = PER_W // CHUNK


def _build_table(embedding_weight, electron_config, config_w, config_b):
    emb_p = (jnp.zeros((TROWS, D), jnp.float32)
             .at[:MAXZ1].set(embedding_weight)
             .at[MAXZ1:2 * MAXZ1].set(embedding_weight))
    ec_p = jnp.zeros((TROWS, 128), jnp.float32).at[:MAXZ1, :NCFG].set(electron_config)
    wt_p = jnp.zeros((128, D), jnp.float32).at[:NCFG].set(config_w.T)
    b_p = jnp.broadcast_to(config_b[None, :], (8, D))
    bmask = (lax.broadcasted_iota(jnp.int32, (TROWS, 1), 0) < MAXZ1).astype(jnp.float32)
    def body(emb_ref, ec_ref, wt_ref, b_ref, m_ref, out_ref):
        elec = jnp.dot(ec_ref[...], wt_ref[...], preferred_element_type=jnp.float32)
        out_ref[...] = emb_ref[...] + elec + b_ref[0:1, :] * m_ref[...]
    return pl.pallas_call(
        body,
        out_shape=jax.ShapeDtypeStruct((TROWS, D), jnp.float32),
    )(emb_p, ec_p, wt_p, b_p, bmask)


PAIR_BI = 16


def _pair_body(t_ref, out_ref):
    i0 = pl.program_id(0) * PAIR_BI
    left = t_ref[pl.ds(i0, PAIR_BI), :]
    out_ref[:, :, 0:D] = jnp.broadcast_to(left[:, None, :], (PAIR_BI, TROWS, D))
    out_ref[:, :, D:2 * D] = jnp.broadcast_to(t_ref[...][None, :, :],
                                              (PAIR_BI, TROWS, D))


def _build_pair_table(table):
    return pl.pallas_call(
        _pair_body,
        grid=(TROWS // PAIR_BI,),
        in_specs=[pl.BlockSpec((TROWS, D), lambda i: (0, 0))],
        out_specs=pl.BlockSpec((PAIR_BI, TROWS, 2 * D), lambda i: (i, 0, 0)),
        out_shape=jax.ShapeDtypeStruct((TROWS, TROWS, 2 * D), jnp.float32),
    )(table).reshape(PROWS, 2 * D)


def _pair_idx_body(even_ref, odd_ref, out_ref):
    col = lax.broadcasted_iota(jnp.int32, even_ref.shape, 1)
    adj_even = even_ref[...] + jnp.where(col == 0, MAXZ1, 0)
    out_ref[...] = adj_even * TROWS + odd_ref[...]


def _build_pair_indices(atomic_numbers):
    even = lax.slice(atomic_numbers, (0, 0), (B, S), (1, 2))
    odd = lax.slice(atomic_numbers, (0, 1), (B, S), (1, 2))
    rows = 512
    return pl.pallas_call(
        _pair_idx_body,
        grid=(B // rows,),
        in_specs=[pl.BlockSpec((rows, S // 2), lambda i: (i, 0)),
                  pl.BlockSpec((rows, S // 2), lambda i: (i, 0))],
        out_specs=pl.BlockSpec((rows, S // 2), lambda i: (i, 0)),
        out_shape=jax.ShapeDtypeStruct((B, S // 2), jnp.int32),
    )(even, odd)


def _gather_body(ptable_hbm, idx_hbm, out_hbm, idx_v,
                 buf0, buf1, gsem0, gsem1, wsem0, wsem1):
    wid = lax.axis_index("s") * NC + lax.axis_index("c")
    base = wid * PER_W
    pltpu.sync_copy(idx_hbm.at[pl.ds(base, PER_W)], idx_v)
    bufs = (buf0, buf1)
    gsems = (gsem0, gsem1)
    wsems = (wsem0, wsem1)

    def body(k, carry):
        handles = []
        for b in range(2):
            c = 2 * k + b
            @pl.when(k > 0)
            def _drain():
                pltpu.make_async_copy(
                    bufs[b], out_hbm.at[pl.ds(base, CHUNK)], wsems[b]).wait()
            handles.append(pltpu.async_copy(
                ptable_hbm.at[idx_v.at[pl.ds(c * CHUNK, CHUNK)]],
                bufs[b], gsems[b]))
        for b in range(2):
            c = 2 * k + b
            handles[b].wait()
            pltpu.async_copy(bufs[b],
                             out_hbm.at[pl.ds(base + c * CHUNK, CHUNK)],
                             wsems[b])
        return carry

    lax.fori_loop(0, NCHUNK // 2, body, 0)
    for b in range(2):
        pltpu.make_async_copy(
            bufs[b], out_hbm.at[pl.ds(base, CHUNK)], wsems[b]).wait()


@functools.partial(
    pl.kernel,
    mesh=plsc.VectorSubcoreMesh(core_axis_name="c", subcore_axis_name="s"),
    out_type=jax.ShapeDtypeStruct((NPAIR, 2 * D), jnp.float32),
    scratch_types=[
        pltpu.VMEM((PER_W,), jnp.int32),
        pltpu.VMEM((CHUNK, 2 * D), jnp.float32),
        pltpu.VMEM((CHUNK, 2 * D), jnp.float32),
        pltpu.SemaphoreType.DMA,
        pltpu.SemaphoreType.DMA,
        pltpu.SemaphoreType.DMA,
        pltpu.SemaphoreType.DMA,
    ],
)
def _sc_gather(ptable_hbm, idx_hbm, out_hbm, idx_v,
               buf0, buf1, gsem0, gsem1, wsem0, wsem1):
    _gather_body(ptable_hbm, idx_hbm, out_hbm, idx_v,
                 buf0, buf1, gsem0, gsem1, wsem0, wsem1)


def kernel(atomic_numbers, embedding_weight, electron_config, config_w, config_b):
    table = _build_table(embedding_weight, electron_config, config_w, config_b)
    ptable = _build_pair_table(table)
    pidx = _build_pair_indices(atomic_numbers).reshape(NPAIR)
    out = _sc_gather(ptable, pidx)
    return out.reshape(B, S, D)
